# baseline (speedup 1.0000x reference)
import functools

import jax
import jax.numpy as jnp
from jax import lax
from jax.experimental import pallas as pl
from jax.experimental.pallas import tpu as pltpu
from jax.experimental.pallas import tpu_sc as plsc

N = 10000
NP = 10240
E = 320000
D_IN = 128
DH = 32
DEGW = 16

NC, NS = 2, 16
NW = NC * NS
CH = 128
EPW = 10240
EP = NW * EPW
CPW = EPW // CH
NB = 8
NGRP = CPW // NB
RPT = NP // NS

_MESH = plsc.VectorSubcoreMesh(core_axis_name="c", subcore_axis_name="s")
_PREC = lax.Precision.HIGHEST
_SC_PARAMS = pltpu.CompilerParams(use_tc_tiling_on_sc=False)

_SEG_BYTES = CH * DH * 4
_DEG_BYTES = CH * DEGW * 4
_IDX_BYTES = CH * 4


@functools.partial(
    pl.kernel,
    out_type=(
        jax.ShapeDtypeStruct((NC, NP, DH), jnp.float32),
        jax.ShapeDtypeStruct((NC, NP, DEGW), jnp.float32),
    ),
    mesh=_MESH,
    scratch_types=[
        pltpu.VMEM((NB, CH), jnp.int32),
        pltpu.VMEM((NB, CH), jnp.int32),
        pltpu.VMEM((NB, CH, DH), jnp.float32),
        pltpu.VMEM((CH, DEGW), jnp.float32),
        pltpu.VMEM_SHARED((NP, DH), jnp.float32),
        pltpu.VMEM_SHARED((NP, DEGW), jnp.float32),
        [pltpu.SemaphoreType.DMA] * NB,
        [pltpu.SemaphoreType.DMA] * NB,
        [pltpu.SemaphoreType.DMA] * NB,
    ],
    compiler_params=_SC_PARAMS,
)
def _sc_segsum_deg(z_hbm, srcm_hbm, dstm_hbm, zs_hbm, zd_hbm, ones_hbm,
                   s_out, deg_out,
                   src_v, dst_v, rows_v, ones_v, s_sh, deg_sh,
                   sem_i, sem_g, sem_s):
    cid = lax.axis_index("c")
    sid = lax.axis_index("s")
    wid = sid * NC + cid
    row0 = wid * CPW

    pltpu.sync_copy(zs_hbm, s_sh.at[pl.ds(sid * RPT, RPT)])
    pltpu.sync_copy(zd_hbm, deg_sh.at[pl.ds(sid * RPT, RPT)])
    pltpu.sync_copy(ones_hbm, ones_v)
    for b in range(NB):
        pltpu.async_copy(srcm_hbm.at[row0 + b], src_v.at[b], sem_i[b])
        pltpu.async_copy(dstm_hbm.at[row0 + b], dst_v.at[b], sem_i[b])
    plsc.subcore_barrier()

    def group(g, carry):
        for b in range(NB):
            pltpu.make_async_copy(srcm_hbm.at[row0], src_v.at[b], sem_i[b]).wait()
            pltpu.make_async_copy(dstm_hbm.at[row0], dst_v.at[b], sem_i[b]).wait()
            pltpu.async_copy(z_hbm.at[src_v.at[b]], rows_v.at[b], sem_g[b])
        for b in range(NB):
            pltpu.make_async_copy(z_hbm.at[src_v.at[b]], rows_v.at[b], sem_g[b]).wait()
            pltpu.async_copy(rows_v.at[b], s_sh.at[dst_v.at[b]], sem_s[b], add=True)
            pltpu.async_copy(ones_v, deg_sh.at[dst_v.at[b]], sem_s[b], add=True)
        for b in range(NB):
            pltpu.make_async_copy(rows_v.at[b], s_sh.at[dst_v.at[b]], sem_s[b]).wait()
            pltpu.make_async_copy(ones_v, deg_sh.at[dst_v.at[b]], sem_s[b]).wait()
            rown = row0 + lax.rem((g + 1) * NB + b, CPW)
            pltpu.async_copy(srcm_hbm.at[rown], src_v.at[b], sem_i[b])
            pltpu.async_copy(dstm_hbm.at[rown], dst_v.at[b], sem_i[b])
        return carry

    lax.fori_loop(0, NGRP, group, 0)
    for b in range(NB):
        pltpu.make_async_copy(srcm_hbm.at[row0], src_v.at[b], sem_i[b]).wait()
        pltpu.make_async_copy(dstm_hbm.at[row0], dst_v.at[b], sem_i[b]).wait()
    plsc.subcore_barrier()

    pltpu.sync_copy(s_sh.at[pl.ds(sid * RPT, RPT)],
                    s_out.at[cid, pl.ds(sid * RPT, RPT)])
    pltpu.sync_copy(deg_sh.at[pl.ds(sid * RPT, RPT)],
                    deg_out.at[cid, pl.ds(sid * RPT, RPT)])


@functools.partial(
    pl.kernel,
    out_type=jax.ShapeDtypeStruct((NC, NP, DH), jnp.float32),
    mesh=_MESH,
    scratch_types=[
        pltpu.VMEM((NB, CH), jnp.int32),
        pltpu.VMEM((NB, CH), jnp.int32),
        pltpu.VMEM((NB, CH, DH), jnp.float32),
        pltpu.VMEM_SHARED((NP, DH), jnp.float32),
        [pltpu.SemaphoreType.DMA] * NB,
        [pltpu.SemaphoreType.DMA] * NB,
        [pltpu.SemaphoreType.DMA] * NB,
    ],
    compiler_params=_SC_PARAMS,
)
def _sc_segsum(z_hbm, srcm_hbm, dstm_hbm, zs_hbm,
               s_out,
               src_v, dst_v, rows_v, s_sh,
               sem_i, sem_g, sem_s):
    cid = lax.axis_index("c")
    sid = lax.axis_index("s")
    wid = sid * NC + cid
    row0 = wid * CPW

    pltpu.sync_copy(zs_hbm, s_sh.at[pl.ds(sid * RPT, RPT)])
    for b in range(NB):
        pltpu.async_copy(srcm_hbm.at[row0 + b], src_v.at[b], sem_i[b])
        pltpu.async_copy(dstm_hbm.at[row0 + b], dst_v.at[b], sem_i[b])
    plsc.subcore_barrier()

    def group(g, carry):
        for b in range(NB):
            pltpu.make_async_copy(srcm_hbm.at[row0], src_v.at[b], sem_i[b]).wait()
            pltpu.make_async_copy(dstm_hbm.at[row0], dst_v.at[b], sem_i[b]).wait()
            pltpu.async_copy(z_hbm.at[src_v.at[b]], rows_v.at[b], sem_g[b])
        for b in range(NB):
            pltpu.make_async_copy(z_hbm.at[src_v.at[b]], rows_v.at[b], sem_g[b]).wait()
            pltpu.async_copy(rows_v.at[b], s_sh.at[dst_v.at[b]], sem_s[b], add=True)
        for b in range(NB):
            pltpu.make_async_copy(rows_v.at[b], s_sh.at[dst_v.at[b]], sem_s[b]).wait()
            rown = row0 + lax.rem((g + 1) * NB + b, CPW)
            pltpu.async_copy(srcm_hbm.at[rown], src_v.at[b], sem_i[b])
            pltpu.async_copy(dstm_hbm.at[rown], dst_v.at[b], sem_i[b])
        return carry

    lax.fori_loop(0, NGRP, group, 0)
    for b in range(NB):
        pltpu.make_async_copy(srcm_hbm.at[row0], src_v.at[b], sem_i[b]).wait()
        pltpu.make_async_copy(dstm_hbm.at[row0], dst_v.at[b], sem_i[b]).wait()
    plsc.subcore_barrier()

    pltpu.sync_copy(s_sh.at[pl.ds(sid * RPT, RPT)],
                    s_out.at[cid, pl.ds(sid * RPT, RPT)])


def _tc_proj_body(x_ref, w_ref, z_ref):
    z_ref[...] = lax.dot_general(
        x_ref[...], w_ref[...], (((1,), (0,)), ((), ())),
        preferred_element_type=jnp.float32, precision=_PREC)


def _tc_mid_body(z1_ref, sp_ref, dp_ref, b1_ref, w2t_ref, z2_ref, invd_ref):
    s = sp_ref[0, 0:N] + sp_ref[1, 0:N]
    deg = dp_ref[0, 0:N, 0:1] + dp_ref[1, 0:N, 0:1]
    invd = 1.0 / jnp.maximum(deg, 1.0)
    h1 = jnp.maximum(z1_ref[...] + b1_ref[...] + s * invd, 0.0)
    z2_ref[...] = lax.dot_general(
        h1, w2t_ref[...], (((1,), (0,)), ((), ())),
        preferred_element_type=jnp.float32, precision=_PREC)
    invd_ref[...] = invd


def _tc_out_body(z2_ref, sp_ref, invd_ref, b2_ref, o_ref):
    s = sp_ref[0, 0:N] + sp_ref[1, 0:N]
    o_ref[...] = jnp.maximum(z2_ref[...] + b2_ref[...] + s * invd_ref[...], 0.0)


def kernel(x, edge_index, W1, b1, W2, b2):
    src = edge_index[0].astype(jnp.int32)
    dst = edge_index[1].astype(jnp.int32)
    pad = EP - E
    srcm = jnp.concatenate([src, jnp.zeros((pad,), jnp.int32)]).reshape(EP // CH, CH)
    dstm = jnp.concatenate([dst, jnp.full((pad,), N, jnp.int32)]).reshape(EP // CH, CH)
    w1t = W1.T
    w2t = W2.T
    b1r = b1.reshape(1, DH)
    b2r = b2.reshape(1, DH)
    zs = jnp.zeros((RPT, DH), jnp.float32)
    zd = jnp.zeros((RPT, DEGW), jnp.float32)
    ones = jnp.ones((CH, DEGW), jnp.float32)

    z1 = pl.pallas_call(
        _tc_proj_body,
        out_shape=jax.ShapeDtypeStruct((N, DH), jnp.float32),
    )(x, w1t)

    s1, degp = _sc_segsum_deg(z1, srcm, dstm, zs, zd, ones)

    z2, invd = pl.pallas_call(
        _tc_mid_body,
        out_shape=(jax.ShapeDtypeStruct((N, DH), jnp.float32),
                   jax.ShapeDtypeStruct((N, 1), jnp.float32)),
    )(z1, s1, degp, b1r, w2t)

    s2 = _sc_segsum(z2, srcm, dstm, zs)

    out = pl.pallas_call(
        _tc_out_body,
        out_shape=jax.ShapeDtypeStruct((N, DH), jnp.float32),
    )(z2, s2, invd, b2r)

    return out


# speedup vs baseline: 1.0250x; 1.0250x over previous
import functools

import jax
import jax.numpy as jnp
from jax import lax
from jax.experimental import pallas as pl
from jax.experimental.pallas import tpu as pltpu
from jax.experimental.pallas import tpu_sc as plsc

N = 10000
NP = 10240
E = 320000
D_IN = 128
DH = 32
DEGW = 16

NC, NS = 2, 16
NW = NC * NS
CH = 128
EPW = 10240
EP = NW * EPW
CPW = EPW // CH
NB = 8
NGRP = CPW // NB
RPT = NP // NS

_MESH = plsc.VectorSubcoreMesh(core_axis_name="c", subcore_axis_name="s")
_PREC = lax.Precision.HIGHEST
_SC_PARAMS = pltpu.CompilerParams(use_tc_tiling_on_sc=False, needs_layout_passes=False)

_SEG_BYTES = CH * DH * 4
_DEG_BYTES = CH * DEGW * 4
_IDX_BYTES = CH * 4


@functools.partial(
    pl.kernel,
    out_type=(
        jax.ShapeDtypeStruct((NC, NP, DH), jnp.float32),
        jax.ShapeDtypeStruct((NC, NS, NP), jnp.float32),
    ),
    mesh=_MESH,
    scratch_types=[
        pltpu.VMEM((NB, CH), jnp.int32),
        pltpu.VMEM((NB, CH), jnp.int32),
        pltpu.VMEM((NB, CH, DH), jnp.float32),
        pltpu.VMEM((NP,), jnp.float32),
        pltpu.VMEM_SHARED((NP, DH), jnp.float32),
        [pltpu.SemaphoreType.DMA] * NB,
        [pltpu.SemaphoreType.DMA] * NB,
        [pltpu.SemaphoreType.DMA] * NB,
    ],
    compiler_params=_SC_PARAMS,
)
def _sc_segsum_deg(z_hbm, srcm_hbm, dstm_hbm, zs_hbm,
                   s_out, deg_out,
                   src_v, dst_v, rows_v, deg_l, s_sh,
                   sem_i, sem_g, sem_s):
    cid = lax.axis_index("c")
    sid = lax.axis_index("s")
    wid = sid * NC + cid
    row0 = wid * CPW

    pltpu.sync_copy(zs_hbm, s_sh.at[pl.ds(sid * RPT, RPT)])

    def zero_deg(j, carry):
        deg_l[pl.ds(j * 16, 16)] = jnp.zeros((16,), jnp.float32)
        return carry

    lax.fori_loop(0, NP // 16, zero_deg, 0)
    for b in range(NB):
        pltpu.async_copy(srcm_hbm.at[row0 + b], src_v.at[b], sem_i[b])
        pltpu.async_copy(dstm_hbm.at[row0 + b], dst_v.at[b], sem_i[b])
    plsc.subcore_barrier()

    def group(g, carry):
        for b in range(NB):
            pltpu.make_async_copy(srcm_hbm.at[row0], src_v.at[b], sem_i[b]).wait()
            pltpu.make_async_copy(dstm_hbm.at[row0], dst_v.at[b], sem_i[b]).wait()
            pltpu.async_copy(z_hbm.at[src_v.at[b]], rows_v.at[b], sem_g[b])
            for j in range(CH // 16):
                idx16 = dst_v[b, pl.ds(j * 16, 16)]
                plsc.addupdate_scatter(deg_l, [idx16], jnp.ones((16,), jnp.float32))
        for b in range(NB):
            pltpu.make_async_copy(z_hbm.at[src_v.at[b]], rows_v.at[b], sem_g[b]).wait()
            pltpu.async_copy(rows_v.at[b], s_sh.at[dst_v.at[b]], sem_s[b], add=True)
        for b in range(NB):
            pltpu.make_async_copy(rows_v.at[b], s_sh.at[dst_v.at[b]], sem_s[b]).wait()
            rown = row0 + lax.rem((g + 1) * NB + b, CPW)
            pltpu.async_copy(srcm_hbm.at[rown], src_v.at[b], sem_i[b])
            pltpu.async_copy(dstm_hbm.at[rown], dst_v.at[b], sem_i[b])
        return carry

    lax.fori_loop(0, NGRP, group, 0)
    for b in range(NB):
        pltpu.make_async_copy(srcm_hbm.at[row0], src_v.at[b], sem_i[b]).wait()
        pltpu.make_async_copy(dstm_hbm.at[row0], dst_v.at[b], sem_i[b]).wait()
    plsc.subcore_barrier()

    pltpu.sync_copy(s_sh.at[pl.ds(sid * RPT, RPT)],
                    s_out.at[cid, pl.ds(sid * RPT, RPT)])
    pltpu.sync_copy(deg_l, deg_out.at[cid, sid])


@functools.partial(
    pl.kernel,
    out_type=jax.ShapeDtypeStruct((NC, NP, DH), jnp.float32),
    mesh=_MESH,
    scratch_types=[
        pltpu.VMEM((NB, CH), jnp.int32),
        pltpu.VMEM((NB, CH), jnp.int32),
        pltpu.VMEM((NB, CH, DH), jnp.float32),
        pltpu.VMEM_SHARED((NP, DH), jnp.float32),
        [pltpu.SemaphoreType.DMA] * NB,
        [pltpu.SemaphoreType.DMA] * NB,
        [pltpu.SemaphoreType.DMA] * NB,
    ],
    compiler_params=_SC_PARAMS,
)
def _sc_segsum(z_hbm, srcm_hbm, dstm_hbm, zs_hbm,
               s_out,
               src_v, dst_v, rows_v, s_sh,
               sem_i, sem_g, sem_s):
    cid = lax.axis_index("c")
    sid = lax.axis_index("s")
    wid = sid * NC + cid
    row0 = wid * CPW

    pltpu.sync_copy(zs_hbm, s_sh.at[pl.ds(sid * RPT, RPT)])
    for b in range(NB):
        pltpu.async_copy(srcm_hbm.at[row0 + b], src_v.at[b], sem_i[b])
        pltpu.async_copy(dstm_hbm.at[row0 + b], dst_v.at[b], sem_i[b])
    plsc.subcore_barrier()

    def group(g, carry):
        for b in range(NB):
            pltpu.make_async_copy(srcm_hbm.at[row0], src_v.at[b], sem_i[b]).wait()
            pltpu.make_async_copy(dstm_hbm.at[row0], dst_v.at[b], sem_i[b]).wait()
            pltpu.async_copy(z_hbm.at[src_v.at[b]], rows_v.at[b], sem_g[b])
        for b in range(NB):
            pltpu.make_async_copy(z_hbm.at[src_v.at[b]], rows_v.at[b], sem_g[b]).wait()
            pltpu.async_copy(rows_v.at[b], s_sh.at[dst_v.at[b]], sem_s[b], add=True)
        for b in range(NB):
            pltpu.make_async_copy(rows_v.at[b], s_sh.at[dst_v.at[b]], sem_s[b]).wait()
            rown = row0 + lax.rem((g + 1) * NB + b, CPW)
            pltpu.async_copy(srcm_hbm.at[rown], src_v.at[b], sem_i[b])
            pltpu.async_copy(dstm_hbm.at[rown], dst_v.at[b], sem_i[b])
        return carry

    lax.fori_loop(0, NGRP, group, 0)
    for b in range(NB):
        pltpu.make_async_copy(srcm_hbm.at[row0], src_v.at[b], sem_i[b]).wait()
        pltpu.make_async_copy(dstm_hbm.at[row0], dst_v.at[b], sem_i[b]).wait()
    plsc.subcore_barrier()

    pltpu.sync_copy(s_sh.at[pl.ds(sid * RPT, RPT)],
                    s_out.at[cid, pl.ds(sid * RPT, RPT)])


def _tc_proj_body(x_ref, w_ref, z_ref):
    z_ref[...] = lax.dot_general(
        x_ref[...], w_ref[...], (((1,), (0,)), ((), ())),
        preferred_element_type=jnp.float32, precision=_PREC)


def _tc_mid_body(z1_ref, sp_ref, dp_ref, b1_ref, w2t_ref, z2_ref, invd_ref):
    s = sp_ref[0, 0:N] + sp_ref[1, 0:N]
    deg = jnp.sum(dp_ref[...].reshape(NC * NS, NP), axis=0)[0:N].reshape(N, 1)
    invd = 1.0 / jnp.maximum(deg, 1.0)
    h1 = jnp.maximum(z1_ref[...] + b1_ref[...] + s * invd, 0.0)
    z2_ref[...] = lax.dot_general(
        h1, w2t_ref[...], (((1,), (0,)), ((), ())),
        preferred_element_type=jnp.float32, precision=_PREC)
    invd_ref[...] = invd


def _tc_out_body(z2_ref, sp_ref, invd_ref, b2_ref, o_ref):
    s = sp_ref[0, 0:N] + sp_ref[1, 0:N]
    o_ref[...] = jnp.maximum(z2_ref[...] + b2_ref[...] + s * invd_ref[...], 0.0)


def kernel(x, edge_index, W1, b1, W2, b2):
    src = edge_index[0].astype(jnp.int32)
    dst = edge_index[1].astype(jnp.int32)
    pad = EP - E
    srcm = jnp.concatenate([src, jnp.zeros((pad,), jnp.int32)]).reshape(EP // CH, CH)
    dstm = jnp.concatenate([dst, jnp.full((pad,), N, jnp.int32)]).reshape(EP // CH, CH)
    w1t = W1.T
    w2t = W2.T
    b1r = b1.reshape(1, DH)
    b2r = b2.reshape(1, DH)
    zs = jnp.zeros((RPT, DH), jnp.float32)

    z1 = pl.pallas_call(
        _tc_proj_body,
        out_shape=jax.ShapeDtypeStruct((N, DH), jnp.float32),
    )(x, w1t)

    s1, degp = _sc_segsum_deg(z1, srcm, dstm, zs)

    z2, invd = pl.pallas_call(
        _tc_mid_body,
        out_shape=(jax.ShapeDtypeStruct((N, DH), jnp.float32),
                   jax.ShapeDtypeStruct((N, 1), jnp.float32)),
    )(z1, s1, degp, b1r, w2t)

    s2 = _sc_segsum(z2, srcm, dstm, zs)

    out = pl.pallas_call(
        _tc_out_body,
        out_shape=jax.ShapeDtypeStruct((N, DH), jnp.float32),
    )(z2, s2, invd, b2r)

    return out


# speedup vs baseline: 2.2807x; 2.2250x over previous
import functools

import jax
import jax.numpy as jnp
from jax import lax
from jax.experimental import pallas as pl
from jax.experimental.pallas import tpu as pltpu
from jax.experimental.pallas import tpu_sc as plsc

N = 10000
NP = 10240
E = 320000
D_IN = 128
DH = 32
DEGW = 16

NC, NS = 2, 16
NW = NC * NS
CH = 80
EPW = E // NW
CPW = EPW // CH
NB = 5
NGRP = CPW // NB
RPT = NP // NS

_MESH = plsc.VectorSubcoreMesh(core_axis_name="c", subcore_axis_name="s")
_PREC = lax.Precision.HIGHEST
_SC_PARAMS = pltpu.CompilerParams(use_tc_tiling_on_sc=False, needs_layout_passes=False)

_SEG_BYTES = CH * DH * 4
_DEG_BYTES = CH * DEGW * 4
_IDX_BYTES = CH * 4


@functools.partial(
    pl.kernel,
    out_type=(
        jax.ShapeDtypeStruct((NC, NP, DH), jnp.float32),
        jax.ShapeDtypeStruct((NC, NP), jnp.float32),
    ),
    mesh=_MESH,
    scratch_types=[
        pltpu.VMEM((NB, CH), jnp.int32),
        pltpu.VMEM((NB, CH), jnp.int32),
        pltpu.VMEM((NB, CH, DH), jnp.float32),
        pltpu.VMEM((NP,), jnp.float32),
        pltpu.VMEM_SHARED((NP, DH), jnp.float32),
        pltpu.VMEM_SHARED((NP, DH), jnp.float32),
        pltpu.VMEM((NS, RPT), jnp.float32),
        pltpu.VMEM((RPT,), jnp.float32),
        pltpu.VMEM_SHARED((NS, NP), jnp.float32),
        [pltpu.SemaphoreType.DMA] * NB,
        [pltpu.SemaphoreType.DMA] * NB,
        [pltpu.SemaphoreType.DMA] * NB,
    ],
    compiler_params=_SC_PARAMS,
)
def _sc_segsum_deg(z_hbm, srcm_hbm, dstm_hbm, zs_hbm,
                   s_out, deg_out,
                   src_v, dst_v, rows_v, deg_l, s_sh, z_sh, deg_t, deg_r, deg_sh,
                   sem_i, sem_g, sem_s):
    cid = lax.axis_index("c")
    sid = lax.axis_index("s")
    wid = sid * NC + cid
    row0 = wid * CPW

    pltpu.sync_copy(zs_hbm, s_sh.at[pl.ds(sid * RPT, RPT)])
    pltpu.sync_copy(z_hbm.at[pl.ds(sid * RPT, RPT)],
                    z_sh.at[pl.ds(sid * RPT, RPT)])

    def zero_deg(j, carry):
        deg_l[pl.ds(j * 16, 16)] = jnp.zeros((16,), jnp.float32)
        return carry

    lax.fori_loop(0, NP // 16, zero_deg, 0)
    for b in range(NB):
        pltpu.async_copy(srcm_hbm.at[pl.ds((row0 + b) * CH, CH)], src_v.at[b], sem_i[b])
        pltpu.async_copy(dstm_hbm.at[pl.ds((row0 + b) * CH, CH)], dst_v.at[b], sem_i[b])
    plsc.subcore_barrier()

    def group(g, carry):
        for b in range(NB):
            pltpu.make_async_copy(srcm_hbm.at[pl.ds(row0 * CH, CH)], src_v.at[b], sem_i[b]).wait()
            pltpu.make_async_copy(dstm_hbm.at[pl.ds(row0 * CH, CH)], dst_v.at[b], sem_i[b]).wait()
            pltpu.async_copy(z_sh.at[src_v.at[b]], rows_v.at[b], sem_g[b])
            for j in range(CH // 16):
                idx16 = dst_v[b, pl.ds(j * 16, 16)]
                plsc.addupdate_scatter(deg_l, [idx16], jnp.ones((16,), jnp.float32))
        for b in range(NB):
            pltpu.make_async_copy(z_sh.at[src_v.at[b]], rows_v.at[b], sem_g[b]).wait()
            pltpu.async_copy(rows_v.at[b], s_sh.at[dst_v.at[b]], sem_s[b], add=True)
        for b in range(NB):
            pltpu.make_async_copy(rows_v.at[b], s_sh.at[dst_v.at[b]], sem_s[b]).wait()
            rown = row0 + lax.rem((g + 1) * NB + b, CPW)
            pltpu.async_copy(srcm_hbm.at[pl.ds(rown * CH, CH)], src_v.at[b], sem_i[b])
            pltpu.async_copy(dstm_hbm.at[pl.ds(rown * CH, CH)], dst_v.at[b], sem_i[b])
        return carry

    lax.fori_loop(0, NGRP, group, 0)
    for b in range(NB):
        pltpu.make_async_copy(srcm_hbm.at[pl.ds(row0 * CH, CH)], src_v.at[b], sem_i[b]).wait()
        pltpu.make_async_copy(dstm_hbm.at[pl.ds(row0 * CH, CH)], dst_v.at[b], sem_i[b]).wait()
    plsc.subcore_barrier()

    pltpu.sync_copy(deg_l, deg_sh.at[sid])
    plsc.subcore_barrier()
    pltpu.sync_copy(s_sh.at[pl.ds(sid * RPT, RPT)],
                    s_out.at[cid, pl.ds(sid * RPT, RPT)])
    pltpu.sync_copy(deg_sh.at[:, pl.ds(sid * RPT, RPT)], deg_t)

    def red(j, carry):
        acc = deg_t[0, pl.ds(j * 16, 16)]
        for r in range(1, NS):
            acc = acc + deg_t[r, pl.ds(j * 16, 16)]
        deg_r[pl.ds(j * 16, 16)] = acc
        return carry

    lax.fori_loop(0, RPT // 16, red, 0)
    pltpu.sync_copy(deg_r, deg_out.at[cid, pl.ds(sid * RPT, RPT)])


@functools.partial(
    pl.kernel,
    out_type=jax.ShapeDtypeStruct((NC, NP, DH), jnp.float32),
    mesh=_MESH,
    scratch_types=[
        pltpu.VMEM((NB, CH), jnp.int32),
        pltpu.VMEM((NB, CH), jnp.int32),
        pltpu.VMEM((NB, CH, DH), jnp.float32),
        pltpu.VMEM_SHARED((NP, DH), jnp.float32),
        pltpu.VMEM_SHARED((NP, DH), jnp.float32),
        [pltpu.SemaphoreType.DMA] * NB,
        [pltpu.SemaphoreType.DMA] * NB,
        [pltpu.SemaphoreType.DMA] * NB,
    ],
    compiler_params=_SC_PARAMS,
)
def _sc_segsum(z_hbm, srcm_hbm, dstm_hbm, zs_hbm,
               s_out,
               src_v, dst_v, rows_v, s_sh, z_sh,
               sem_i, sem_g, sem_s):
    cid = lax.axis_index("c")
    sid = lax.axis_index("s")
    wid = sid * NC + cid
    row0 = wid * CPW

    pltpu.sync_copy(zs_hbm, s_sh.at[pl.ds(sid * RPT, RPT)])
    pltpu.sync_copy(z_hbm.at[pl.ds(sid * RPT, RPT)],
                    z_sh.at[pl.ds(sid * RPT, RPT)])
    for b in range(NB):
        pltpu.async_copy(srcm_hbm.at[pl.ds((row0 + b) * CH, CH)], src_v.at[b], sem_i[b])
        pltpu.async_copy(dstm_hbm.at[pl.ds((row0 + b) * CH, CH)], dst_v.at[b], sem_i[b])
    plsc.subcore_barrier()

    def group(g, carry):
        for b in range(NB):
            pltpu.make_async_copy(srcm_hbm.at[pl.ds(row0 * CH, CH)], src_v.at[b], sem_i[b]).wait()
            pltpu.make_async_copy(dstm_hbm.at[pl.ds(row0 * CH, CH)], dst_v.at[b], sem_i[b]).wait()
            pltpu.async_copy(z_sh.at[src_v.at[b]], rows_v.at[b], sem_g[b])
        for b in range(NB):
            pltpu.make_async_copy(z_sh.at[src_v.at[b]], rows_v.at[b], sem_g[b]).wait()
            pltpu.async_copy(rows_v.at[b], s_sh.at[dst_v.at[b]], sem_s[b], add=True)
        for b in range(NB):
            pltpu.make_async_copy(rows_v.at[b], s_sh.at[dst_v.at[b]], sem_s[b]).wait()
            rown = row0 + lax.rem((g + 1) * NB + b, CPW)
            pltpu.async_copy(srcm_hbm.at[pl.ds(rown * CH, CH)], src_v.at[b], sem_i[b])
            pltpu.async_copy(dstm_hbm.at[pl.ds(rown * CH, CH)], dst_v.at[b], sem_i[b])
        return carry

    lax.fori_loop(0, NGRP, group, 0)
    for b in range(NB):
        pltpu.make_async_copy(srcm_hbm.at[pl.ds(row0 * CH, CH)], src_v.at[b], sem_i[b]).wait()
        pltpu.make_async_copy(dstm_hbm.at[pl.ds(row0 * CH, CH)], dst_v.at[b], sem_i[b]).wait()
    plsc.subcore_barrier()

    pltpu.sync_copy(s_sh.at[pl.ds(sid * RPT, RPT)],
                    s_out.at[cid, pl.ds(sid * RPT, RPT)])


def _tc_proj_body(x_ref, ei_ref, w_ref, z_ref, src_ref, dst_ref):
    z_ref[0:N] = lax.dot_general(
        x_ref[...], w_ref[...], (((1,), (1,)), ((), ())),
        preferred_element_type=jnp.float32)
    z_ref[pl.ds(N, NP - N)] = jnp.zeros((NP - N, DH), jnp.float32)
    src_ref[...] = ei_ref[0]
    dst_ref[...] = ei_ref[1]


def _tc_mid_body(z1_ref, sp_ref, dp_ref, b1_ref, w2t_ref, z2_ref, invd_ref):
    s = sp_ref[0, 0:N] + sp_ref[1, 0:N]
    invd = 1.0 / jnp.maximum(dp_ref[0, 0:N] + dp_ref[1, 0:N], 1.0)
    h1 = jnp.maximum(z1_ref[0:N] + b1_ref[...] + s * invd.reshape(N, 1), 0.0)
    z2_ref[0:N] = lax.dot_general(
        h1, w2t_ref[...], (((1,), (1,)), ((), ())),
        preferred_element_type=jnp.float32)
    z2_ref[pl.ds(N, NP - N)] = jnp.zeros((NP - N, DH), jnp.float32)
    invd_ref[...] = invd


def _tc_out_body(z2_ref, sp_ref, invd_ref, b2_ref, o_ref):
    s = sp_ref[0, 0:N] + sp_ref[1, 0:N]
    o_ref[...] = jnp.maximum(
        z2_ref[0:N] + b2_ref[...] + s * invd_ref[...].reshape(N, 1), 0.0)


def kernel(x, edge_index, W1, b1, W2, b2):
    ei = edge_index.astype(jnp.int32)
    b1r = b1.reshape(1, DH)
    b2r = b2.reshape(1, DH)
    zs = jnp.zeros((RPT, DH), jnp.float32)

    z1, srcm, dstm = pl.pallas_call(
        _tc_proj_body, name="tc_proj",
        out_shape=(jax.ShapeDtypeStruct((NP, DH), jnp.float32),
                   jax.ShapeDtypeStruct((E,), jnp.int32),
                   jax.ShapeDtypeStruct((E,), jnp.int32)),
    )(x, ei, W1)

    s1, degp = _sc_segsum_deg(z1, srcm, dstm, zs)

    z2, invd = pl.pallas_call(
        _tc_mid_body, name="tc_mid",
        out_shape=(jax.ShapeDtypeStruct((NP, DH), jnp.float32),
                   jax.ShapeDtypeStruct((N,), jnp.float32)),
    )(z1, s1, degp, b1r, W2)

    s2 = _sc_segsum(z2, srcm, dstm, zs)

    out = pl.pallas_call(
        _tc_out_body, name="tc_out",
        out_shape=jax.ShapeDtypeStruct((N, DH), jnp.float32),
    )(z2, s2, invd, b2r)

    return out
